# Initial kernel scaffold; baseline (speedup 1.0000x reference)
import jax
import jax.numpy as jnp
from jax.experimental import pallas as pl


def kernel(x, edge_index, batch_index, W1, as1, ad1, b1, W2, as2, ad2, b2, W3, as3, ad3, b3):
    raise NotImplementedError("write your pallas kernel here")


# speedup vs baseline: 19.7190x; 19.7190x over previous
import functools

import jax
import jax.numpy as jnp
from jax import lax
from jax.experimental import pallas as pl
from jax.experimental.pallas import tpu as pltpu
from jax.experimental.pallas import tpu_sc as plsc

N = 10000
E = 320000
NUM_GRAPHS = 16

NC, NS, L = 2, 16, 16
NW = NC * NS

NP = 10240
EP = 360448
ROWS = EP // 128
RPT_A = ROWS // NW
RPT_B = ROWS // NS
NPT = NP // NS
LIVE_ROWS = (E + N + 127) // 128
SB = 16


def _mesh():
  return plsc.VectorSubcoreMesh(core_axis_name="c", subcore_axis_name="s")


def _leaky(v):
  return jnp.where(v > 0, v, 0.2 * v)


def _tc_body(nchunks, *refs):
  i = pl.program_id(0)
  *chunk_refs, b_ref, w_ref, asv_ref, adv_ref = refs[:-4]
  h_ref, als_ref, ald_ref, amax_ref = refs[-4:]
  if nchunks == 0:
    act = chunk_refs[0][...]
  else:
    act = jnp.concatenate([r[...] for r in chunk_refs], axis=1)
    act = jax.nn.relu(act + b_ref[...])
  h = jnp.dot(act, w_ref[...], preferred_element_type=jnp.float32)
  h_ref[...] = h
  als = jnp.dot(h, asv_ref[...], preferred_element_type=jnp.float32)
  ald = jnp.dot(h, adv_ref[...], preferred_element_type=jnp.float32)
  als_ref[...] = als
  ald_ref[...] = ald
  cur = jnp.max(als).reshape(1, 1)

  @pl.when(i == 0)
  def _():
    amax_ref[...] = cur

  @pl.when(i > 0)
  def _():
    amax_ref[...] = jnp.maximum(amax_ref[...], cur)


def _tc_layer(chunks, b, w, asv, adv):
  din = sum(c.shape[1] for c in chunks)
  dout = w.shape[1]
  R = 1000
  grid = (N // R,)
  nchunks = 0 if b is None else len(chunks)
  if b is None:
    b_arr = jnp.zeros((1, din), jnp.float32)
  else:
    b_arr = b.reshape(1, din)
  in_specs = (
      [pl.BlockSpec((R, c.shape[1]), lambda i: (i, 0)) for c in chunks]
      + [
          pl.BlockSpec((1, din), lambda i: (0, 0)),
          pl.BlockSpec((din, dout), lambda i: (0, 0)),
          pl.BlockSpec((dout, 1), lambda i: (0, 0)),
          pl.BlockSpec((dout, 1), lambda i: (0, 0)),
      ]
  )
  out_specs = [
      pl.BlockSpec((R, dout), lambda i: (i, 0)),
      pl.BlockSpec((R, 1), lambda i: (i, 0)),
      pl.BlockSpec((R, 1), lambda i: (i, 0)),
      pl.BlockSpec((1, 1), lambda i: (0, 0)),
  ]
  out_shape = [
      jax.ShapeDtypeStruct((N, dout), jnp.float32),
      jax.ShapeDtypeStruct((N, 1), jnp.float32),
      jax.ShapeDtypeStruct((N, 1), jnp.float32),
      jax.ShapeDtypeStruct((1, 1), jnp.float32),
  ]
  body = functools.partial(_tc_body, nchunks)
  return pl.pallas_call(
      body,
      grid=grid,
      in_specs=in_specs,
      out_specs=out_specs,
      out_shape=out_shape,
  )(*chunks, b_arr, w, asv.reshape(dout, 1), adv.reshape(dout, 1))


def _tc_final(parts, b3, batch_index):
  def body(parts_ref, b_ref, bi_ref, pooled_ref, acc_ref):
    i = pl.program_id(0)

    @pl.when(i == 0)
    def _():
      acc_ref[...] = parts_ref[0]

    @pl.when(i > 0)
    def _():
      acc_ref[...] = acc_ref[...] + parts_ref[0]

    @pl.when(i == NW - 1)
    def _():
      h3 = jax.nn.relu(acc_ref[:, :N] + b_ref[...])
      gids = lax.broadcasted_iota(jnp.int32, (NUM_GRAPHS, N), 0)
      onehot = jnp.where(bi_ref[...] == gids, 1.0, 0.0)
      sums = lax.dot_general(
          onehot, h3, (((1,), (1,)), ((), ())),
          preferred_element_type=jnp.float32)
      counts = jnp.sum(onehot, axis=1, keepdims=True)
      pooled_ref[...] = sums / jnp.maximum(counts, 1.0)

  return pl.pallas_call(
      body,
      grid=(NW,),
      in_specs=[
          pl.BlockSpec((1, 4, NP), lambda i: (i, 0, 0)),
          pl.BlockSpec((4, 1), lambda i: (0, 0)),
          pl.BlockSpec((1, N), lambda i: (0, 0)),
      ],
      out_specs=pl.BlockSpec((NUM_GRAPHS, 4), lambda i: (0, 0)),
      out_shape=jax.ShapeDtypeStruct((NUM_GRAPHS, 4), jnp.float32),
      scratch_shapes=[pltpu.VMEM((4, NP), jnp.float32)],
  )(parts, b3.reshape(4, 1), batch_index.reshape(1, N))


def _sc_edge_p(src2d, dst2d, als, ald, amax16, iota2d):
  @functools.partial(
      pl.kernel,
      out_type=(
          jax.ShapeDtypeStruct((ROWS, 128), jnp.float32),
          jax.ShapeDtypeStruct((NC * NP,), jnp.float32),
      ),
      mesh=_mesh(),
      compiler_params=pltpu.CompilerParams(needs_layout_passes=False),
      scratch_types=dict(
          als_t=pltpu.VMEM((NP,), jnp.float32),
          ald_t=pltpu.VMEM((NP,), jnp.float32),
          am_t=pltpu.VMEM((16,), jnp.float32),
          s_t=pltpu.VMEM((NP,), jnp.float32),
          src_t=pltpu.VMEM((RPT_A, 128), jnp.int32),
          dst_t=pltpu.VMEM((RPT_A, 128), jnp.int32),
          p_t=pltpu.VMEM((RPT_A, 128), jnp.float32),
          iota_t=pltpu.VMEM((NP // 128, 128), jnp.int32),
          z_t=pltpu.VMEM((NPT,), jnp.float32),
          s_sh=pltpu.VMEM_SHARED((NP,), jnp.float32),
      ),
  )
  def k(src_h, dst_h, als_h, ald_h, am_h, iota_h, p_h, sparts_h, *, als_t,
        ald_t, am_t, s_t, src_t, dst_t, p_t, iota_t, z_t, s_sh):
    cid = lax.axis_index("c")
    sid = lax.axis_index("s")
    wid = sid * NC + cid

    pltpu.sync_copy(als_h, als_t)
    pltpu.sync_copy(ald_h, ald_t)
    pltpu.sync_copy(am_h, am_t)
    pltpu.sync_copy(iota_h, iota_t)
    pltpu.sync_copy(src_h.at[pl.ds(wid * RPT_A, RPT_A)], src_t)
    pltpu.sync_copy(dst_h.at[pl.ds(wid * RPT_A, RPT_A)], dst_t)

    zv = jnp.zeros((L,), jnp.float32)

    def zloop(i, _):
      s_t[pl.ds(i * L, L)] = zv
      return 0

    lax.fori_loop(0, NP // L, zloop, 0)

    def zloop2(i, _):
      z_t[pl.ds(i * L, L)] = zv
      return 0

    lax.fori_loop(0, NPT // L, zloop2, 0)
    pltpu.sync_copy(z_t, s_sh.at[pl.ds(sid * NPT, NPT)])

    am = am_t[pl.ds(0, L)][0]

    def row(g, _):
      @pl.when(wid * RPT_A + g < LIVE_ROWS)
      def _():
        def grp(j, _):
          sl = pl.ds(j * L, L)
          sv = src_t[g, sl]
          dv = dst_t[g, sl]
          a1 = plsc.load_gather(als_t, [sv])
          a2 = plsc.load_gather(ald_t, [dv])
          e = _leaky(a1 + a2)
          mb = _leaky(am + a2)
          pv = jnp.exp(e - mb)
          p_t[g, sl] = pv
          plsc.addupdate_scatter(s_t, [dv], pv)
          return 0

        lax.fori_loop(0, 128 // L, grp, 0)
      return 0

    lax.fori_loop(0, RPT_A, row, 0)
    pltpu.sync_copy(p_t, p_h.at[pl.ds(wid * RPT_A, RPT_A)])

    plsc.subcore_barrier()

    def sred(g, _):
      pltpu.sync_copy(
          s_t.at[pl.ds(g * 128, 128)], s_sh.at[iota_t.at[g]], add=True)
      return 0

    lax.fori_loop(0, NP // 128, sred, 0)
    plsc.subcore_barrier()
    pltpu.sync_copy(
        s_sh.at[pl.ds(sid * NPT, NPT)],
        sparts_h.at[pl.ds(cid * NP + sid * NPT, NPT)])

  return k(src2d, dst2d, als, ald, amax16, iota2d)


def _sc_alpha(dst2d, p2d, s_parts):
  @functools.partial(
      pl.kernel,
      out_type=jax.ShapeDtypeStruct((ROWS, 128), jnp.float32),
      mesh=_mesh(),
      compiler_params=pltpu.CompilerParams(needs_layout_passes=False),
      scratch_types=dict(
          s_t=pltpu.VMEM((NP,), jnp.float32),
          tmp_t=pltpu.VMEM((NP,), jnp.float32),
          dst_t=pltpu.VMEM((RPT_A, 128), jnp.int32),
          p_t=pltpu.VMEM((RPT_A, 128), jnp.float32),
          a_t=pltpu.VMEM((RPT_A, 128), jnp.float32),
      ),
  )
  def k(dst_h, p_h, sparts_h, alpha_h, *, s_t, tmp_t, dst_t, p_t, a_t):
    cid = lax.axis_index("c")
    sid = lax.axis_index("s")
    wid = sid * NC + cid
    pltpu.sync_copy(sparts_h.at[pl.ds(0, NP)], s_t)
    pltpu.sync_copy(sparts_h.at[pl.ds(NP, NP)], tmp_t)

    def addl(i, _):
      sl = pl.ds(i * L, L)
      s_t[sl] = s_t[sl] + tmp_t[sl]
      return 0

    lax.fori_loop(0, NP // L, addl, 0)
    pltpu.sync_copy(dst_h.at[pl.ds(wid * RPT_A, RPT_A)], dst_t)
    pltpu.sync_copy(p_h.at[pl.ds(wid * RPT_A, RPT_A)], p_t)

    def row(g, _):
      @pl.when(wid * RPT_A + g < LIVE_ROWS)
      def _():
        def grp(j, _):
          sl = pl.ds(j * L, L)
          dv = dst_t[g, sl]
          sg = plsc.load_gather(s_t, [dv])
          a_t[g, sl] = p_t[g, sl] / (sg + 1e-16)
          return 0

        lax.fori_loop(0, 128 // L, grp, 0)
      return 0

    lax.fori_loop(0, RPT_A, row, 0)
    pltpu.sync_copy(a_t, alpha_h.at[pl.ds(wid * RPT_A, RPT_A)])

  return k(dst2d, p2d, s_parts)


def _sc_aggregate(src2d, dst2d, alpha2d, h_chunks):
  nch = len(h_chunks)

  @functools.partial(
      pl.kernel,
      out_type=tuple(
          jax.ShapeDtypeStruct((NP, 128), jnp.float32) for _ in range(nch)),
      mesh=_mesh(),
      compiler_params=pltpu.CompilerParams(needs_layout_passes=False),
      scratch_types=dict(
          src_t=pltpu.VMEM((SB, 128), jnp.int32),
          dst_t=pltpu.VMEM((SB, 128), jnp.int32),
          a_t=pltpu.VMEM((SB, 128), jnp.float32),
          rows_t=pltpu.VMEM((128, 128), jnp.float32),
          z_t=pltpu.VMEM((SB, 128), jnp.float32),
          sem=pltpu.SemaphoreType.DMA,
          acc_sh=pltpu.VMEM_SHARED((NP, 128), jnp.float32),
      ),
  )
  def k(src_h, dst_h, alpha_h, *rest, src_t, dst_t, a_t, rows_t, z_t, sem,
        acc_sh):
    h_hs = rest[:nch]
    out_hs = rest[nch:]
    cid = lax.axis_index("c")
    sid = lax.axis_index("s")

    zv = jnp.zeros((L,), jnp.float32)

    def zrow(i, _):
      for j in range(128 // L):
        z_t[i, pl.ds(j * L, L)] = zv
      return 0

    lax.fori_loop(0, SB, zrow, 0)

    for c in range(nch):

      @pl.when(c % NC == cid)
      def _():
        def zacc(i, _):
          pltpu.sync_copy(z_t, acc_sh.at[pl.ds((sid * (NPT // SB) + i) * SB,
                                               SB)])
          return 0

        lax.fori_loop(0, NPT // SB, zacc, 0)
        plsc.subcore_barrier()

        def block(bb, _):
          row0 = sid * RPT_B + bb * SB

          @pl.when(row0 < LIVE_ROWS)
          def _():
            pltpu.sync_copy(src_h.at[pl.ds(row0, SB)], src_t)
            pltpu.sync_copy(dst_h.at[pl.ds(row0, SB)], dst_t)
            pltpu.sync_copy(alpha_h.at[pl.ds(row0, SB)], a_t)

            def row(g, _):
              @pl.when(row0 + g < LIVE_ROWS)
              def _():
                _agg_row(g)
              return 0

            def _agg_row(g):
              pltpu.async_copy(h_hs[c].at[src_t.at[g]], rows_t, sem).wait()

              def scale(j8, _):
                avv = a_t[g, pl.ds(j8 * L, L)]
                for kk in range(L):
                  av = jnp.full((L,), avv[kk], jnp.float32)
                  r = j8 * L + kk
                  for j in range(128 // L):
                    sl = pl.ds(j * L, L)
                    rows_t[r, sl] = rows_t[r, sl] * av
                return 0

              lax.fori_loop(0, 128 // L, scale, 0)
              pltpu.sync_copy(rows_t, acc_sh.at[dst_t.at[g]], add=True)
              return 0

            lax.fori_loop(0, SB, row, 0)
          return 0

        lax.fori_loop(0, RPT_B // SB, block, 0)
        plsc.subcore_barrier()
        pltpu.sync_copy(
            acc_sh.at[pl.ds(sid * NPT, NPT)],
            out_hs[c].at[pl.ds(sid * NPT, NPT)])
        plsc.subcore_barrier()

  return k(src2d, dst2d, alpha2d, *h_chunks)


def _sc_aggregate_small(src2d, dst2d, alpha2d, h3t):
  @functools.partial(
      pl.kernel,
      out_type=jax.ShapeDtypeStruct((NW, 4, NP), jnp.float32),
      mesh=_mesh(),
      compiler_params=pltpu.CompilerParams(needs_layout_passes=False),
      scratch_types=dict(
          h_t=pltpu.VMEM((4, NP), jnp.float32),
          acc_t=pltpu.VMEM((4, NP), jnp.float32),
          src_t=pltpu.VMEM((RPT_A, 128), jnp.int32),
          dst_t=pltpu.VMEM((RPT_A, 128), jnp.int32),
          a_t=pltpu.VMEM((RPT_A, 128), jnp.float32),
      ),
  )
  def k(src_h, dst_h, alpha_h, h3_h, parts_h, *, h_t, acc_t, src_t, dst_t,
        a_t):
    cid = lax.axis_index("c")
    sid = lax.axis_index("s")
    wid = sid * NC + cid
    pltpu.sync_copy(h3_h, h_t)
    pltpu.sync_copy(src_h.at[pl.ds(wid * RPT_A, RPT_A)], src_t)
    pltpu.sync_copy(dst_h.at[pl.ds(wid * RPT_A, RPT_A)], dst_t)
    pltpu.sync_copy(alpha_h.at[pl.ds(wid * RPT_A, RPT_A)], a_t)

    zv = jnp.zeros((L,), jnp.float32)

    def zloop(i, _):
      for j in range(4):
        acc_t[j, pl.ds(i * L, L)] = zv
      return 0

    lax.fori_loop(0, NP // L, zloop, 0)

    def row(g, _):
      @pl.when(wid * RPT_A + g < LIVE_ROWS)
      def _():
        def grp(j, _):
          sl = pl.ds(j * L, L)
          sv = src_t[g, sl]
          dv = dst_t[g, sl]
          av = a_t[g, sl]
          for col in range(4):
            cv = jnp.full((L,), col, jnp.int32)
            hv = plsc.load_gather(h_t, [cv, sv])
            plsc.addupdate_scatter(acc_t, [cv, dv], hv * av)
          return 0

        lax.fori_loop(0, 128 // L, grp, 0)
      return 0

    lax.fori_loop(0, RPT_A, row, 0)
    pltpu.sync_copy(acc_t, parts_h.at[wid])

  return k(src2d, dst2d, alpha2d, h3t)


def kernel(x, edge_index, batch_index, W1, as1, ad1, b1, W2, as2, ad2, b2,
           W3, as3, ad3, b3):
  loops = jnp.arange(N, dtype=edge_index.dtype)
  src = jnp.concatenate([edge_index[0], loops])
  dst = jnp.concatenate([edge_index[1], loops])
  npad = EP - (E + N)
  src = jnp.concatenate([src, jnp.zeros((npad,), jnp.int32)])
  dst = jnp.concatenate([dst, jnp.full((npad,), N, jnp.int32)])
  src2d = src.reshape(ROWS, 128)
  dst2d = dst.reshape(ROWS, 128)
  iota2d = jnp.arange(NP, dtype=jnp.int32).reshape(NP // 128, 128)

  def pad_nodes(v):
    return jnp.concatenate([v.reshape(N), jnp.zeros((NP - N,), jnp.float32)])

  def edge_phase(h, als, ald, amax):
    als_p = pad_nodes(als)
    ald_p = pad_nodes(ald)
    am16 = jnp.broadcast_to(amax.reshape(1), (16,))
    p2d, s_parts = _sc_edge_p(src2d, dst2d, als_p, ald_p, am16, iota2d)
    alpha2d = _sc_alpha(dst2d, p2d, s_parts)
    return alpha2d

  h1, als1, ald1, am1 = _tc_layer([x], None, W1, as1, ad1)
  alpha1 = edge_phase(h1, als1, ald1, am1)
  h1_chunks = [h1[:, c * 128:(c + 1) * 128] for c in range(2)]
  o1 = _sc_aggregate(src2d, dst2d, alpha1, h1_chunks)
  o1 = [o[:N] for o in o1]

  h2, als2, ald2, am2 = _tc_layer(o1, b1, W2, as2, ad2)
  alpha2 = edge_phase(h2, als2, ald2, am2)
  h2_chunks = [h2[:, c * 128:(c + 1) * 128] for c in range(4)]
  o2 = _sc_aggregate(src2d, dst2d, alpha2, h2_chunks)
  o2 = [o[:N] for o in o2]

  h3, als3, ald3, am3 = _tc_layer(o2, b2, W3, as3, ad3)
  alpha3 = edge_phase(h3, als3, ald3, am3)
  h3t = jnp.pad(h3, ((0, NP - N), (0, 0))).T
  parts = _sc_aggregate_small(src2d, dst2d, alpha3, h3t)

  return _tc_final(parts, b3, batch_index)


# speedup vs baseline: 28.2435x; 1.4323x over previous
import functools

import jax
import jax.numpy as jnp
from jax import lax
from jax.experimental import pallas as pl
from jax.experimental.pallas import tpu as pltpu
from jax.experimental.pallas import tpu_sc as plsc

N = 10000
E = 320000
NUM_GRAPHS = 16

NC, NS, L = 2, 16, 16
NW = NC * NS

NP = 10240
EP = 360448
ROWS = EP // 128
RPT_A = ROWS // NW
RPT_B = ROWS // NS
NPT = NP // NS
LIVE_ROWS = (E + N + 127) // 128
SB = 16


def _mesh():
  return plsc.VectorSubcoreMesh(core_axis_name="c", subcore_axis_name="s")


def _leaky(v):
  return jnp.where(v > 0, v, 0.2 * v)


def _tc_body(nchunks, *refs):
  i = pl.program_id(0)
  *chunk_refs, b_ref, w_ref, asv_ref, adv_ref = refs[:-4]
  h_ref, als_ref, ald_ref, amax_ref = refs[-4:]
  if nchunks == 0:
    act = chunk_refs[0][...]
  else:
    act = jnp.concatenate([r[...] for r in chunk_refs], axis=1)
    act = jax.nn.relu(act + b_ref[...])
  h = jnp.dot(act, w_ref[...], preferred_element_type=jnp.float32)
  h_ref[...] = h
  als = jnp.dot(h, asv_ref[...], preferred_element_type=jnp.float32)
  ald = jnp.dot(h, adv_ref[...], preferred_element_type=jnp.float32)
  als_ref[...] = als
  ald_ref[...] = ald
  cur = jnp.max(als).reshape(1, 1)

  @pl.when(i == 0)
  def _():
    amax_ref[...] = cur

  @pl.when(i > 0)
  def _():
    amax_ref[...] = jnp.maximum(amax_ref[...], cur)


def _tc_layer(chunks, b, w, asv, adv):
  din = sum(c.shape[1] for c in chunks)
  dout = w.shape[1]
  R = 1000
  grid = (N // R,)
  nchunks = 0 if b is None else len(chunks)
  if b is None:
    b_arr = jnp.zeros((1, din), jnp.float32)
  else:
    b_arr = b.reshape(1, din)
  in_specs = (
      [pl.BlockSpec((R, c.shape[1]), lambda i: (i, 0)) for c in chunks]
      + [
          pl.BlockSpec((1, din), lambda i: (0, 0)),
          pl.BlockSpec((din, dout), lambda i: (0, 0)),
          pl.BlockSpec((dout, 1), lambda i: (0, 0)),
          pl.BlockSpec((dout, 1), lambda i: (0, 0)),
      ]
  )
  out_specs = [
      pl.BlockSpec((R, dout), lambda i: (i, 0)),
      pl.BlockSpec((R, 1), lambda i: (i, 0)),
      pl.BlockSpec((R, 1), lambda i: (i, 0)),
      pl.BlockSpec((1, 1), lambda i: (0, 0)),
  ]
  out_shape = [
      jax.ShapeDtypeStruct((N, dout), jnp.float32),
      jax.ShapeDtypeStruct((N, 1), jnp.float32),
      jax.ShapeDtypeStruct((N, 1), jnp.float32),
      jax.ShapeDtypeStruct((1, 1), jnp.float32),
  ]
  body = functools.partial(_tc_body, nchunks)
  return pl.pallas_call(
      body,
      grid=grid,
      in_specs=in_specs,
      out_specs=out_specs,
      out_shape=out_shape,
  )(*chunks, b_arr, w, asv.reshape(dout, 1), adv.reshape(dout, 1))


def _tc_final(parts, b3, batch_index):
  def body(parts_ref, b_ref, bi_ref, pooled_ref, acc_ref):
    i = pl.program_id(0)

    @pl.when(i == 0)
    def _():
      acc_ref[...] = parts_ref[0]

    @pl.when(i > 0)
    def _():
      acc_ref[...] = acc_ref[...] + parts_ref[0]

    @pl.when(i == NW - 1)
    def _():
      h3 = jax.nn.relu(acc_ref[:, :N] + b_ref[...])
      gids = lax.broadcasted_iota(jnp.int32, (NUM_GRAPHS, N), 0)
      onehot = jnp.where(bi_ref[...] == gids, 1.0, 0.0)
      sums = lax.dot_general(
          onehot, h3, (((1,), (1,)), ((), ())),
          preferred_element_type=jnp.float32)
      counts = jnp.sum(onehot, axis=1, keepdims=True)
      pooled_ref[...] = sums / jnp.maximum(counts, 1.0)

  return pl.pallas_call(
      body,
      grid=(NW,),
      in_specs=[
          pl.BlockSpec((1, 4, NP), lambda i: (i, 0, 0)),
          pl.BlockSpec((4, 1), lambda i: (0, 0)),
          pl.BlockSpec((1, N), lambda i: (0, 0)),
      ],
      out_specs=pl.BlockSpec((NUM_GRAPHS, 4), lambda i: (0, 0)),
      out_shape=jax.ShapeDtypeStruct((NUM_GRAPHS, 4), jnp.float32),
      scratch_shapes=[pltpu.VMEM((4, NP), jnp.float32)],
  )(parts, b3.reshape(4, 1), batch_index.reshape(1, N))


def _sc_edge_p(src2d, dst2d, als, ald, amax16, iota2d):
  @functools.partial(
      pl.kernel,
      out_type=(
          jax.ShapeDtypeStruct((ROWS, 128), jnp.float32),
          jax.ShapeDtypeStruct((NC * NP,), jnp.float32),
      ),
      mesh=_mesh(),
      compiler_params=pltpu.CompilerParams(needs_layout_passes=False),
      scratch_types=dict(
          als_t=pltpu.VMEM((NP,), jnp.float32),
          ald_t=pltpu.VMEM((NP,), jnp.float32),
          am_t=pltpu.VMEM((16,), jnp.float32),
          s_t=pltpu.VMEM((NP,), jnp.float32),
          src_t=pltpu.VMEM((RPT_A, 128), jnp.int32),
          dst_t=pltpu.VMEM((RPT_A, 128), jnp.int32),
          p_t=pltpu.VMEM((RPT_A, 128), jnp.float32),
          iota_t=pltpu.VMEM((NP // 128, 128), jnp.int32),
          z_t=pltpu.VMEM((NPT,), jnp.float32),
          s_sh=pltpu.VMEM_SHARED((NP,), jnp.float32),
      ),
  )
  def k(src_h, dst_h, als_h, ald_h, am_h, iota_h, p_h, sparts_h, *, als_t,
        ald_t, am_t, s_t, src_t, dst_t, p_t, iota_t, z_t, s_sh):
    cid = lax.axis_index("c")
    sid = lax.axis_index("s")
    wid = sid * NC + cid

    pltpu.sync_copy(als_h, als_t)
    pltpu.sync_copy(ald_h, ald_t)
    pltpu.sync_copy(am_h, am_t)
    pltpu.sync_copy(iota_h, iota_t)
    pltpu.sync_copy(src_h.at[pl.ds(wid * RPT_A, RPT_A)], src_t)
    pltpu.sync_copy(dst_h.at[pl.ds(wid * RPT_A, RPT_A)], dst_t)

    zv = jnp.zeros((L,), jnp.float32)

    def zloop(i, _):
      s_t[pl.ds(i * L, L)] = zv
      return 0

    lax.fori_loop(0, NP // L, zloop, 0)

    def zloop2(i, _):
      z_t[pl.ds(i * L, L)] = zv
      return 0

    lax.fori_loop(0, NPT // L, zloop2, 0)
    pltpu.sync_copy(z_t, s_sh.at[pl.ds(sid * NPT, NPT)])

    am = am_t[pl.ds(0, L)][0]

    def row(g, _):
      @pl.when(wid * RPT_A + g < LIVE_ROWS)
      def _():
        def grp(j, _):
          sl = pl.ds(j * L, L)
          sv = src_t[g, sl]
          dv = dst_t[g, sl]
          a1 = plsc.load_gather(als_t, [sv])
          a2 = plsc.load_gather(ald_t, [dv])
          e = _leaky(a1 + a2)
          mb = _leaky(am + a2)
          pv = jnp.exp(e - mb)
          p_t[g, sl] = pv
          plsc.addupdate_scatter(s_t, [dv], pv)
          return 0

        lax.fori_loop(0, 128 // L, grp, 0)
      return 0

    lax.fori_loop(0, RPT_A, row, 0)
    pltpu.sync_copy(p_t, p_h.at[pl.ds(wid * RPT_A, RPT_A)])

    plsc.subcore_barrier()

    def sred(g, _):
      pltpu.sync_copy(
          s_t.at[pl.ds(g * 128, 128)], s_sh.at[iota_t.at[g]], add=True)
      return 0

    lax.fori_loop(0, NP // 128, sred, 0)
    plsc.subcore_barrier()
    pltpu.sync_copy(
        s_sh.at[pl.ds(sid * NPT, NPT)],
        sparts_h.at[pl.ds(cid * NP + sid * NPT, NPT)])

  return k(src2d, dst2d, als, ald, amax16, iota2d)


def _sc_alpha(dst2d, p2d, s_parts):
  @functools.partial(
      pl.kernel,
      out_type=jax.ShapeDtypeStruct((ROWS, 128), jnp.float32),
      mesh=_mesh(),
      compiler_params=pltpu.CompilerParams(needs_layout_passes=False),
      scratch_types=dict(
          s_t=pltpu.VMEM((NP,), jnp.float32),
          tmp_t=pltpu.VMEM((NP,), jnp.float32),
          dst_t=pltpu.VMEM((RPT_A, 128), jnp.int32),
          p_t=pltpu.VMEM((RPT_A, 128), jnp.float32),
          a_t=pltpu.VMEM((RPT_A, 128), jnp.float32),
      ),
  )
  def k(dst_h, p_h, sparts_h, alpha_h, *, s_t, tmp_t, dst_t, p_t, a_t):
    cid = lax.axis_index("c")
    sid = lax.axis_index("s")
    wid = sid * NC + cid
    pltpu.sync_copy(sparts_h.at[pl.ds(0, NP)], s_t)
    pltpu.sync_copy(sparts_h.at[pl.ds(NP, NP)], tmp_t)

    def addl(i, _):
      sl = pl.ds(i * L, L)
      s_t[sl] = s_t[sl] + tmp_t[sl]
      return 0

    lax.fori_loop(0, NP // L, addl, 0)
    pltpu.sync_copy(dst_h.at[pl.ds(wid * RPT_A, RPT_A)], dst_t)
    pltpu.sync_copy(p_h.at[pl.ds(wid * RPT_A, RPT_A)], p_t)

    def row(g, _):
      @pl.when(wid * RPT_A + g < LIVE_ROWS)
      def _():
        def grp(j, _):
          sl = pl.ds(j * L, L)
          dv = dst_t[g, sl]
          sg = plsc.load_gather(s_t, [dv])
          a_t[g, sl] = p_t[g, sl] / (sg + 1e-16)
          return 0

        lax.fori_loop(0, 128 // L, grp, 0)
      return 0

    lax.fori_loop(0, RPT_A, row, 0)
    pltpu.sync_copy(a_t, alpha_h.at[pl.ds(wid * RPT_A, RPT_A)])

  return k(dst2d, p2d, s_parts)


def _sc_aggregate(src2d, dst2d, alpha2d, h_chunks):
  nch = len(h_chunks)

  @functools.partial(
      pl.kernel,
      out_type=tuple(
          jax.ShapeDtypeStruct((NP, 128), jnp.float32) for _ in range(nch)),
      mesh=_mesh(),
      compiler_params=pltpu.CompilerParams(needs_layout_passes=False),
      scratch_types=dict(
          src_t=pltpu.VMEM((SB, 128), jnp.int32),
          dst_t=pltpu.VMEM((SB, 128), jnp.int32),
          a_t=pltpu.VMEM((SB, 128), jnp.float32),
          rows_t=pltpu.VMEM((128, 128), jnp.float32),
          rows2_t=pltpu.VMEM((128, 128), jnp.float32),
          z_t=pltpu.VMEM((SB, 128), jnp.float32),
          sem=pltpu.SemaphoreType.DMA,
          sem2=pltpu.SemaphoreType.DMA,
          acc_sh=pltpu.VMEM_SHARED((NP, 128), jnp.float32),
      ),
  )
  def k(src_h, dst_h, alpha_h, *rest, src_t, dst_t, a_t, rows_t, rows2_t,
        z_t, sem, sem2, acc_sh):
    h_hs = rest[:nch]
    out_hs = rest[nch:]
    cid = lax.axis_index("c")
    sid = lax.axis_index("s")

    zv = jnp.zeros((L,), jnp.float32)

    def zrow(i, _):
      for j in range(128 // L):
        z_t[i, pl.ds(j * L, L)] = zv
      return 0

    lax.fori_loop(0, SB, zrow, 0)

    for c in range(nch):

      @pl.when(c % NC == cid)
      def _():
        def zacc(i, _):
          pltpu.sync_copy(z_t, acc_sh.at[pl.ds((sid * (NPT // SB) + i) * SB,
                                               SB)])
          return 0

        lax.fori_loop(0, NPT // SB, zacc, 0)
        plsc.subcore_barrier()

        def _scale(buf, g):
          def scale(j8, _):
            avv = a_t[g, pl.ds(j8 * L, L)]
            for kk in range(L):
              av = jnp.full((L,), avv[kk], jnp.float32)
              r = j8 * L + kk
              for j in range(128 // L):
                sl = pl.ds(j * L, L)
                buf[r, sl] = buf[r, sl] * av
            return 0

          lax.fori_loop(0, 128 // L, scale, 0)

        def block(bb, _):
          row0 = sid * RPT_B + bb * SB

          @pl.when(row0 < LIVE_ROWS)
          def _():
            pltpu.sync_copy(src_h.at[pl.ds(row0, SB)], src_t)
            pltpu.sync_copy(dst_h.at[pl.ds(row0, SB)], dst_t)
            pltpu.sync_copy(alpha_h.at[pl.ds(row0, SB)], a_t)

          @pl.when(row0 + SB <= LIVE_ROWS)
          def _():
            pltpu.make_async_copy(
                h_hs[c].at[src_t.at[0]], rows_t, sem).start()

            def pair(i, _):
              g0 = i * 2
              g1 = g0 + 1
              pltpu.make_async_copy(
                  h_hs[c].at[src_t.at[g0]], rows_t, sem).wait()
              pltpu.make_async_copy(
                  h_hs[c].at[src_t.at[g1]], rows2_t, sem2).start()
              _scale(rows_t, g0)
              pltpu.sync_copy(rows_t, acc_sh.at[dst_t.at[g0]], add=True)
              pltpu.make_async_copy(
                  h_hs[c].at[src_t.at[g1]], rows2_t, sem2).wait()

              @pl.when(g1 + 1 < SB)
              def _():
                pltpu.make_async_copy(
                    h_hs[c].at[src_t.at[g1 + 1]], rows_t, sem).start()

              _scale(rows2_t, g1)
              pltpu.sync_copy(rows2_t, acc_sh.at[dst_t.at[g1]], add=True)
              return 0

            lax.fori_loop(0, SB // 2, pair, 0)

          @pl.when(
              jnp.logical_and(row0 < LIVE_ROWS, row0 + SB > LIVE_ROWS))
          def _():
            def row(g, _):
              @pl.when(row0 + g < LIVE_ROWS)
              def _():
                pltpu.async_copy(h_hs[c].at[src_t.at[g]], rows_t, sem).wait()
                _scale(rows_t, g)
                pltpu.sync_copy(rows_t, acc_sh.at[dst_t.at[g]], add=True)
              return 0

            lax.fori_loop(0, SB, row, 0)
          return 0

        lax.fori_loop(0, RPT_B // SB, block, 0)
        plsc.subcore_barrier()
        pltpu.sync_copy(
            acc_sh.at[pl.ds(sid * NPT, NPT)],
            out_hs[c].at[pl.ds(sid * NPT, NPT)])
        plsc.subcore_barrier()

  return k(src2d, dst2d, alpha2d, *h_chunks)


def _sc_aggregate_small(src2d, dst2d, alpha2d, h3t):
  @functools.partial(
      pl.kernel,
      out_type=jax.ShapeDtypeStruct((NW, 4, NP), jnp.float32),
      mesh=_mesh(),
      compiler_params=pltpu.CompilerParams(needs_layout_passes=False),
      scratch_types=dict(
          h_t=pltpu.VMEM((4, NP), jnp.float32),
          acc_t=pltpu.VMEM((4, NP), jnp.float32),
          src_t=pltpu.VMEM((RPT_A, 128), jnp.int32),
          dst_t=pltpu.VMEM((RPT_A, 128), jnp.int32),
          a_t=pltpu.VMEM((RPT_A, 128), jnp.float32),
      ),
  )
  def k(src_h, dst_h, alpha_h, h3_h, parts_h, *, h_t, acc_t, src_t, dst_t,
        a_t):
    cid = lax.axis_index("c")
    sid = lax.axis_index("s")
    wid = sid * NC + cid
    pltpu.sync_copy(h3_h, h_t)
    pltpu.sync_copy(src_h.at[pl.ds(wid * RPT_A, RPT_A)], src_t)
    pltpu.sync_copy(dst_h.at[pl.ds(wid * RPT_A, RPT_A)], dst_t)
    pltpu.sync_copy(alpha_h.at[pl.ds(wid * RPT_A, RPT_A)], a_t)

    zv = jnp.zeros((L,), jnp.float32)

    def zloop(i, _):
      for j in range(4):
        acc_t[j, pl.ds(i * L, L)] = zv
      return 0

    lax.fori_loop(0, NP // L, zloop, 0)

    def row(g, _):
      @pl.when(wid * RPT_A + g < LIVE_ROWS)
      def _():
        def grp(j, _):
          sl = pl.ds(j * L, L)
          sv = src_t[g, sl]
          dv = dst_t[g, sl]
          av = a_t[g, sl]
          for col in range(4):
            cv = jnp.full((L,), col, jnp.int32)
            hv = plsc.load_gather(h_t, [cv, sv])
            plsc.addupdate_scatter(acc_t, [cv, dv], hv * av)
          return 0

        lax.fori_loop(0, 128 // L, grp, 0)
      return 0

    lax.fori_loop(0, RPT_A, row, 0)
    pltpu.sync_copy(acc_t, parts_h.at[wid])

  return k(src2d, dst2d, alpha2d, h3t)


def kernel(x, edge_index, batch_index, W1, as1, ad1, b1, W2, as2, ad2, b2,
           W3, as3, ad3, b3):
  loops = jnp.arange(N, dtype=edge_index.dtype)
  src = jnp.concatenate([edge_index[0], loops])
  dst = jnp.concatenate([edge_index[1], loops])
  npad = EP - (E + N)
  src = jnp.concatenate([src, jnp.zeros((npad,), jnp.int32)])
  dst = jnp.concatenate([dst, jnp.full((npad,), N, jnp.int32)])
  src2d = src.reshape(ROWS, 128)
  dst2d = dst.reshape(ROWS, 128)
  iota2d = jnp.arange(NP, dtype=jnp.int32).reshape(NP // 128, 128)

  def pad_nodes(v):
    return jnp.concatenate([v.reshape(N), jnp.zeros((NP - N,), jnp.float32)])

  def edge_phase(h, als, ald, amax):
    als_p = pad_nodes(als)
    ald_p = pad_nodes(ald)
    am16 = jnp.broadcast_to(amax.reshape(1), (16,))
    p2d, s_parts = _sc_edge_p(src2d, dst2d, als_p, ald_p, am16, iota2d)
    alpha2d = _sc_alpha(dst2d, p2d, s_parts)
    return alpha2d

  h1, als1, ald1, am1 = _tc_layer([x], None, W1, as1, ad1)
  alpha1 = edge_phase(h1, als1, ald1, am1)
  h1_chunks = [h1[:, c * 128:(c + 1) * 128] for c in range(2)]
  o1 = _sc_aggregate(src2d, dst2d, alpha1, h1_chunks)
  o1 = [o[:N] for o in o1]

  h2, als2, ald2, am2 = _tc_layer(o1, b1, W2, as2, ad2)
  alpha2 = edge_phase(h2, als2, ald2, am2)
  h2_chunks = [h2[:, c * 128:(c + 1) * 128] for c in range(4)]
  o2 = _sc_aggregate(src2d, dst2d, alpha2, h2_chunks)
  o2 = [o[:N] for o in o2]

  h3, als3, ald3, am3 = _tc_layer(o2, b2, W3, as3, ad3)
  alpha3 = edge_phase(h3, als3, ald3, am3)
  h3t = jnp.pad(h3, ((0, NP - N), (0, 0))).T
  parts = _sc_aggregate_small(src2d, dst2d, alpha3, h3t)

  return _tc_final(parts, b3, batch_index)


# speedup vs baseline: 28.5260x; 1.0100x over previous
import functools

import jax
import jax.numpy as jnp
from jax import lax
from jax.experimental import pallas as pl
from jax.experimental.pallas import tpu as pltpu
from jax.experimental.pallas import tpu_sc as plsc

N = 10000
E = 320000
NUM_GRAPHS = 16

NC, NS, L = 2, 16, 16
NW = NC * NS

NP = 10240
SR = NP // 128
EP = 360448
ROWS = EP // 128
RPT_A = ROWS // NW
RPT_B = ROWS // NS
NPT = NP // NS
LIVE_ROWS = (E + N + 127) // 128
SB = 16


def _mesh():
  return plsc.VectorSubcoreMesh(core_axis_name="c", subcore_axis_name="s")


def _leaky(v):
  return jnp.where(v > 0, v, 0.2 * v)


def _tc_body(nchunks, *refs):
  i = pl.program_id(0)
  *chunk_refs, b_ref, w_ref, asv_ref, adv_ref = refs[:-4]
  h_ref, als_ref, ald_ref, amax_ref = refs[-4:]
  if nchunks == 0:
    act = chunk_refs[0][...]
  else:
    act = jnp.concatenate([r[...] for r in chunk_refs], axis=1)
    act = jax.nn.relu(act + b_ref[...])
  h = jnp.dot(act, w_ref[...], preferred_element_type=jnp.float32)
  h_ref[...] = h
  als = jnp.dot(h, asv_ref[...], preferred_element_type=jnp.float32)
  ald = jnp.dot(h, adv_ref[...], preferred_element_type=jnp.float32)
  als_ref[...] = als
  ald_ref[...] = ald
  cur = jnp.max(als).reshape(1, 1)

  @pl.when(i == 0)
  def _():
    amax_ref[...] = cur

  @pl.when(i > 0)
  def _():
    amax_ref[...] = jnp.maximum(amax_ref[...], cur)


def _tc_layer(chunks, b, w, asv, adv):
  din = sum(c.shape[1] for c in chunks)
  dout = w.shape[1]
  R = 1000
  grid = (N // R,)
  nchunks = 0 if b is None else len(chunks)
  if b is None:
    b_arr = jnp.zeros((1, din), jnp.float32)
  else:
    b_arr = b.reshape(1, din)
  in_specs = (
      [pl.BlockSpec((R, c.shape[1]), lambda i: (i, 0)) for c in chunks]
      + [
          pl.BlockSpec((1, din), lambda i: (0, 0)),
          pl.BlockSpec((din, dout), lambda i: (0, 0)),
          pl.BlockSpec((dout, 1), lambda i: (0, 0)),
          pl.BlockSpec((dout, 1), lambda i: (0, 0)),
      ]
  )
  out_specs = [
      pl.BlockSpec((R, dout), lambda i: (i, 0)),
      pl.BlockSpec((R, 1), lambda i: (i, 0)),
      pl.BlockSpec((R, 1), lambda i: (i, 0)),
      pl.BlockSpec((1, 1), lambda i: (0, 0)),
  ]
  out_shape = [
      jax.ShapeDtypeStruct((N, dout), jnp.float32),
      jax.ShapeDtypeStruct((N, 1), jnp.float32),
      jax.ShapeDtypeStruct((N, 1), jnp.float32),
      jax.ShapeDtypeStruct((1, 1), jnp.float32),
  ]
  body = functools.partial(_tc_body, nchunks)
  return pl.pallas_call(
      body,
      grid=grid,
      in_specs=in_specs,
      out_specs=out_specs,
      out_shape=out_shape,
  )(*chunks, b_arr, w, asv.reshape(dout, 1), adv.reshape(dout, 1))


def _tc_final(parts, b3, batch_index):
  def body(parts_ref, b_ref, bi_ref, pooled_ref, acc_ref):
    i = pl.program_id(0)

    @pl.when(i == 0)
    def _():
      acc_ref[...] = parts_ref[0]

    @pl.when(i > 0)
    def _():
      acc_ref[...] = acc_ref[...] + parts_ref[0]

    @pl.when(i == NW - 1)
    def _():
      h3 = jax.nn.relu(acc_ref[:, :N] + b_ref[...])
      gids = lax.broadcasted_iota(jnp.int32, (NUM_GRAPHS, N), 0)
      onehot = jnp.where(bi_ref[...] == gids, 1.0, 0.0)
      sums = lax.dot_general(
          onehot, h3, (((1,), (1,)), ((), ())),
          preferred_element_type=jnp.float32)
      counts = jnp.sum(onehot, axis=1, keepdims=True)
      pooled_ref[...] = sums / jnp.maximum(counts, 1.0)

  return pl.pallas_call(
      body,
      grid=(NW,),
      in_specs=[
          pl.BlockSpec((1, 4, NP), lambda i: (i, 0, 0)),
          pl.BlockSpec((4, 1), lambda i: (0, 0)),
          pl.BlockSpec((1, N), lambda i: (0, 0)),
      ],
      out_specs=pl.BlockSpec((NUM_GRAPHS, 4), lambda i: (0, 0)),
      out_shape=jax.ShapeDtypeStruct((NUM_GRAPHS, 4), jnp.float32),
      scratch_shapes=[pltpu.VMEM((4, NP), jnp.float32)],
  )(parts, b3.reshape(4, 1), batch_index.reshape(1, N))


def _sc_edge_p(src2d, dst2d, als, ald, amax16, irows):
  @functools.partial(
      pl.kernel,
      out_type=(
          jax.ShapeDtypeStruct((ROWS, 128), jnp.float32),
          jax.ShapeDtypeStruct((NC * SR, 128), jnp.float32),
      ),
      mesh=_mesh(),
      compiler_params=pltpu.CompilerParams(needs_layout_passes=False),
      scratch_types=dict(
          als_t=pltpu.VMEM((NP,), jnp.float32),
          ald_t=pltpu.VMEM((NP,), jnp.float32),
          am_t=pltpu.VMEM((16,), jnp.float32),
          s_t=pltpu.VMEM((SR, 128), jnp.float32),
          src_t=pltpu.VMEM((RPT_A, 128), jnp.int32),
          dst_t=pltpu.VMEM((RPT_A, 128), jnp.int32),
          p_t=pltpu.VMEM((RPT_A, 128), jnp.float32),
          irows_t=pltpu.VMEM((1, SR), jnp.int32),
          z8_t=pltpu.VMEM((8, 128), jnp.float32),
          s_sh=pltpu.VMEM_SHARED((SR, 128), jnp.float32),
      ),
  )
  def k(src_h, dst_h, als_h, ald_h, am_h, irows_h, p_h, sparts_h, *, als_t,
        ald_t, am_t, s_t, src_t, dst_t, p_t, irows_t, z8_t, s_sh):
    cid = lax.axis_index("c")
    sid = lax.axis_index("s")
    wid = sid * NC + cid

    pltpu.sync_copy(als_h, als_t)
    pltpu.sync_copy(ald_h, ald_t)
    pltpu.sync_copy(am_h, am_t)
    pltpu.sync_copy(irows_h, irows_t)
    pltpu.sync_copy(src_h.at[pl.ds(wid * RPT_A, RPT_A)], src_t)
    pltpu.sync_copy(dst_h.at[pl.ds(wid * RPT_A, RPT_A)], dst_t)

    zv = jnp.zeros((L,), jnp.float32)

    def zloop(i, _):
      s_t[i >> 3, pl.ds((i & 7) * L, L)] = zv
      return 0

    lax.fori_loop(0, SR * 8, zloop, 0)

    for r8 in range(8):
      for j in range(128 // L):
        z8_t[r8, pl.ds(j * L, L)] = zv

    @pl.when(sid < SR // 8)
    def _():
      pltpu.sync_copy(z8_t, s_sh.at[pl.ds(sid * 8, 8)])

    am = am_t[pl.ds(0, L)][0]

    def row(g, _):
      @pl.when(wid * RPT_A + g < LIVE_ROWS)
      def _():
        def grp(j, _):
          sl = pl.ds(j * L, L)
          sv = src_t[g, sl]
          dv = dst_t[g, sl]
          a1 = plsc.load_gather(als_t, [sv])
          a2 = plsc.load_gather(ald_t, [dv])
          e = _leaky(a1 + a2)
          mb = _leaky(am + a2)
          pv = jnp.exp(e - mb)
          p_t[g, sl] = pv
          plsc.addupdate_scatter(s_t, [dv >> 7, dv & 127], pv)
          return 0

        lax.fori_loop(0, 128 // L, grp, 0)
      return 0

    lax.fori_loop(0, RPT_A, row, 0)
    pltpu.sync_copy(p_t, p_h.at[pl.ds(wid * RPT_A, RPT_A)])

    plsc.subcore_barrier()
    pltpu.sync_copy(s_t, s_sh.at[irows_t.at[0]], add=True)
    plsc.subcore_barrier()

    @pl.when(sid < SR // 8)
    def _():
      pltpu.sync_copy(
          s_sh.at[pl.ds(sid * 8, 8)],
          sparts_h.at[pl.ds(cid * SR + sid * 8, 8)])

  return k(src2d, dst2d, als, ald, amax16, irows)


def _sc_aggregate(src2d, dst2d, p2d, s_parts, h_chunks):
  nch = len(h_chunks)

  @functools.partial(
      pl.kernel,
      out_type=tuple(
          jax.ShapeDtypeStruct((NP, 128), jnp.float32) for _ in range(nch)),
      mesh=_mesh(),
      compiler_params=pltpu.CompilerParams(needs_layout_passes=False),
      scratch_types=dict(
          s_t=pltpu.VMEM((SR, 128), jnp.float32),
          src_t=pltpu.VMEM((SB, 128), jnp.int32),
          dst_t=pltpu.VMEM((SB, 128), jnp.int32),
          a_t=pltpu.VMEM((SB, 128), jnp.float32),
          rows_t=pltpu.VMEM((128, 128), jnp.float32),
          rows2_t=pltpu.VMEM((128, 128), jnp.float32),
          sem=pltpu.SemaphoreType.DMA,
          sem2=pltpu.SemaphoreType.DMA,
          acc_sh=pltpu.VMEM_SHARED((NP, 128), jnp.float32),
      ),
  )
  def k(src_h, dst_h, p_h, sparts_h, *rest, s_t, src_t, dst_t, a_t, rows_t,
        rows2_t, sem, sem2, acc_sh):
    h_hs = rest[:nch]
    out_hs = rest[nch:]
    cid = lax.axis_index("c")
    sid = lax.axis_index("s")

    pltpu.sync_copy(sparts_h.at[pl.ds(0, SR)], s_t)
    pltpu.sync_copy(sparts_h.at[pl.ds(SR, SR)], rows_t.at[pl.ds(0, SR)])

    def addl(i, _):
      g = i >> 3
      sl = pl.ds((i & 7) * L, L)
      s_t[g, sl] = s_t[g, sl] + rows_t[g, sl]
      return 0

    lax.fori_loop(0, SR * 8, addl, 0)

    zv = jnp.zeros((L,), jnp.float32)

    def _scale(buf, g):
      def scale(j8, _):
        avv = a_t[g, pl.ds(j8 * L, L)]
        for kk in range(L):
          av = jnp.full((L,), avv[kk], jnp.float32)
          r = j8 * L + kk
          for j in range(128 // L):
            sl = pl.ds(j * L, L)
            buf[r, sl] = buf[r, sl] * av
        return 0

      lax.fori_loop(0, 128 // L, scale, 0)

    for c in range(nch):

      @pl.when(c % NC == cid)
      def _():
        def zrow(i, _):
          for j in range(128 // L):
            rows2_t[i, pl.ds(j * L, L)] = zv
          return 0

        lax.fori_loop(0, 64, zrow, 0)

        def zacc(i, _):
          pltpu.sync_copy(rows2_t.at[pl.ds(0, 64)],
                          acc_sh.at[pl.ds((sid * (NPT // 64) + i) * 64, 64)])
          return 0

        lax.fori_loop(0, NPT // 64, zacc, 0)
        plsc.subcore_barrier()

        def block(bb, _):
          row0 = sid * RPT_B + bb * SB

          @pl.when(row0 < LIVE_ROWS)
          def _():
            pltpu.sync_copy(src_h.at[pl.ds(row0, SB)], src_t)
            pltpu.sync_copy(dst_h.at[pl.ds(row0, SB)], dst_t)
            pltpu.sync_copy(p_h.at[pl.ds(row0, SB)], a_t)

            def arow(g, _):
              for j in range(128 // L):
                sl = pl.ds(j * L, L)
                dv = dst_t[g, sl]
                sg = plsc.load_gather(s_t, [dv >> 7, dv & 127])
                a_t[g, sl] = a_t[g, sl] / (sg + 1e-16)
              return 0

            lax.fori_loop(0, SB, arow, 0)

          @pl.when(row0 + SB <= LIVE_ROWS)
          def _():
            pltpu.make_async_copy(
                h_hs[c].at[src_t.at[0]], rows_t, sem).start()

            def pair(i, _):
              g0 = i * 2
              g1 = g0 + 1
              pltpu.make_async_copy(
                  h_hs[c].at[src_t.at[g0]], rows_t, sem).wait()
              pltpu.make_async_copy(
                  h_hs[c].at[src_t.at[g1]], rows2_t, sem2).start()
              _scale(rows_t, g0)
              pltpu.sync_copy(rows_t, acc_sh.at[dst_t.at[g0]], add=True)
              pltpu.make_async_copy(
                  h_hs[c].at[src_t.at[g1]], rows2_t, sem2).wait()

              @pl.when(g1 + 1 < SB)
              def _():
                pltpu.make_async_copy(
                    h_hs[c].at[src_t.at[g1 + 1]], rows_t, sem).start()

              _scale(rows2_t, g1)
              pltpu.sync_copy(rows2_t, acc_sh.at[dst_t.at[g1]], add=True)
              return 0

            lax.fori_loop(0, SB // 2, pair, 0)

          @pl.when(
              jnp.logical_and(row0 < LIVE_ROWS, row0 + SB > LIVE_ROWS))
          def _():
            def row(g, _):
              @pl.when(row0 + g < LIVE_ROWS)
              def _():
                pltpu.async_copy(h_hs[c].at[src_t.at[g]], rows_t, sem).wait()
                _scale(rows_t, g)
                pltpu.sync_copy(rows_t, acc_sh.at[dst_t.at[g]], add=True)
              return 0

            lax.fori_loop(0, SB, row, 0)
          return 0

        lax.fori_loop(0, RPT_B // SB, block, 0)
        plsc.subcore_barrier()
        pltpu.sync_copy(
            acc_sh.at[pl.ds(sid * NPT, NPT)],
            out_hs[c].at[pl.ds(sid * NPT, NPT)])
        plsc.subcore_barrier()

  return k(src2d, dst2d, p2d, s_parts, *h_chunks)


def _sc_aggregate_small(src2d, dst2d, p2d, s_parts, h3r):
  @functools.partial(
      pl.kernel,
      out_type=jax.ShapeDtypeStruct((NW, 4 * SR, 128), jnp.float32),
      mesh=_mesh(),
      compiler_params=pltpu.CompilerParams(needs_layout_passes=False),
      scratch_types=dict(
          s_t=pltpu.VMEM((SR, 128), jnp.float32),
          h_t=pltpu.VMEM((4 * SR, 128), jnp.float32),
          acc_t=pltpu.VMEM((4 * SR, 128), jnp.float32),
          src_t=pltpu.VMEM((RPT_A, 128), jnp.int32),
          dst_t=pltpu.VMEM((RPT_A, 128), jnp.int32),
          p_t=pltpu.VMEM((RPT_A, 128), jnp.float32),
      ),
  )
  def k(src_h, dst_h, p_h, sparts_h, h3_h, parts_h, *, s_t, h_t, acc_t,
        src_t, dst_t, p_t):
    cid = lax.axis_index("c")
    sid = lax.axis_index("s")
    wid = sid * NC + cid
    pltpu.sync_copy(sparts_h.at[pl.ds(0, SR)], s_t)
    pltpu.sync_copy(sparts_h.at[pl.ds(SR, SR)], h_t.at[pl.ds(0, SR)])

    def addl(i, _):
      g = i >> 3
      sl = pl.ds((i & 7) * L, L)
      s_t[g, sl] = s_t[g, sl] + h_t[g, sl]
      return 0

    lax.fori_loop(0, SR * 8, addl, 0)

    pltpu.sync_copy(h3_h, h_t)
    pltpu.sync_copy(src_h.at[pl.ds(wid * RPT_A, RPT_A)], src_t)
    pltpu.sync_copy(dst_h.at[pl.ds(wid * RPT_A, RPT_A)], dst_t)
    pltpu.sync_copy(p_h.at[pl.ds(wid * RPT_A, RPT_A)], p_t)

    zv = jnp.zeros((L,), jnp.float32)

    def zloop(i, _):
      acc_t[i >> 3, pl.ds((i & 7) * L, L)] = zv
      return 0

    lax.fori_loop(0, 4 * SR * 8, zloop, 0)

    def row(g, _):
      @pl.when(wid * RPT_A + g < LIVE_ROWS)
      def _():
        def grp(j, _):
          sl = pl.ds(j * L, L)
          sv = src_t[g, sl]
          dv = dst_t[g, sl]
          dv_hi = dv >> 7
          dv_lo = dv & 127
          sv_hi = sv >> 7
          sv_lo = sv & 127
          sg = plsc.load_gather(s_t, [dv_hi, dv_lo])
          av = p_t[g, sl] / (sg + 1e-16)
          for col in range(4):
            hv = plsc.load_gather(h_t, [sv_hi + (col * SR), sv_lo])
            plsc.addupdate_scatter(acc_t, [dv_hi + (col * SR), dv_lo],
                                   hv * av)
          return 0

        lax.fori_loop(0, 128 // L, grp, 0)
      return 0

    lax.fori_loop(0, RPT_A, row, 0)
    pltpu.sync_copy(acc_t, parts_h.at[wid])

  return k(src2d, dst2d, p2d, s_parts, h3r)


def kernel(x, edge_index, batch_index, W1, as1, ad1, b1, W2, as2, ad2, b2,
           W3, as3, ad3, b3):
  loops = jnp.arange(N, dtype=edge_index.dtype)
  src = jnp.concatenate([edge_index[0], loops])
  dst = jnp.concatenate([edge_index[1], loops])
  npad = EP - (E + N)
  src = jnp.concatenate([src, jnp.zeros((npad,), jnp.int32)])
  dst = jnp.concatenate([dst, jnp.full((npad,), N, jnp.int32)])
  src2d = src.reshape(ROWS, 128)
  dst2d = dst.reshape(ROWS, 128)
  irows = jnp.arange(SR, dtype=jnp.int32).reshape(1, SR)

  def pad_nodes(v):
    return jnp.concatenate([v.reshape(N), jnp.zeros((NP - N,), jnp.float32)])

  def edge_phase(als, ald, amax):
    als_p = pad_nodes(als)
    ald_p = pad_nodes(ald)
    am16 = jnp.broadcast_to(amax.reshape(1), (16,))
    return _sc_edge_p(src2d, dst2d, als_p, ald_p, am16, irows)

  h1, als1, ald1, am1 = _tc_layer([x], None, W1, as1, ad1)
  p1, s1 = edge_phase(als1, ald1, am1)
  h1_chunks = [h1[:, c * 128:(c + 1) * 128] for c in range(2)]
  o1 = _sc_aggregate(src2d, dst2d, p1, s1, h1_chunks)
  o1 = [o[:N] for o in o1]

  h2, als2, ald2, am2 = _tc_layer(o1, b1, W2, as2, ad2)
  p2, s2 = edge_phase(als2, ald2, am2)
  h2_chunks = [h2[:, c * 128:(c + 1) * 128] for c in range(4)]
  o2 = _sc_aggregate(src2d, dst2d, p2, s2, h2_chunks)
  o2 = [o[:N] for o in o2]

  h3, als3, ald3, am3 = _tc_layer(o2, b2, W3, as3, ad3)
  p3, s3 = edge_phase(als3, ald3, am3)
  h3r = jnp.pad(h3, ((0, NP - N), (0, 0))).T.reshape(4 * SR, 128)
  parts = _sc_aggregate_small(src2d, dst2d, p3, s3, h3r)
  parts = parts.reshape(NW, 4, NP)

  return _tc_final(parts, b3, batch_index)


# speedup vs baseline: 29.2380x; 1.0250x over previous
import functools

import jax
import jax.numpy as jnp
from jax import lax
from jax.experimental import pallas as pl
from jax.experimental.pallas import tpu as pltpu
from jax.experimental.pallas import tpu_sc as plsc

N = 10000
E = 320000
NUM_GRAPHS = 16

NC, NS, L = 2, 16, 16
NW = NC * NS

NP = 10240
SR = NP // 128
EP = 360448
ROWS = EP // 128
RPT_A = ROWS // NW
RPT_B = ROWS // NS
NPT = NP // NS
LIVE_ROWS = (E + N + 127) // 128
SB = 16
HROWS = EP // 64
HPT = HROWS // NS
HSB = 16
LIVE_H = (E + N + 63) // 64


def _mesh():
  return plsc.VectorSubcoreMesh(core_axis_name="c", subcore_axis_name="s")


def _leaky(v):
  return jnp.where(v > 0, v, 0.2 * v)


def _tc_body(nchunks, *refs):
  i = pl.program_id(0)
  *chunk_refs, b_ref, w_ref, asv_ref, adv_ref = refs[:-4]
  h_ref, als_ref, ald_ref, amax_ref = refs[-4:]
  if nchunks == 0:
    act = chunk_refs[0][...]
  else:
    act = jnp.concatenate([r[...] for r in chunk_refs], axis=1)
    act = jax.nn.relu(act + b_ref[...])
  h = jnp.dot(act, w_ref[...], preferred_element_type=jnp.float32)
  h_ref[...] = h
  als = jnp.dot(h, asv_ref[...], preferred_element_type=jnp.float32)
  ald = jnp.dot(h, adv_ref[...], preferred_element_type=jnp.float32)
  als_ref[...] = als
  ald_ref[...] = ald
  cur = jnp.max(als).reshape(1, 1)

  @pl.when(i == 0)
  def _():
    amax_ref[...] = cur

  @pl.when(i > 0)
  def _():
    amax_ref[...] = jnp.maximum(amax_ref[...], cur)


def _tc_layer(chunks, b, w, asv, adv):
  din = sum(c.shape[1] for c in chunks)
  dout = w.shape[1]
  R = 1000
  grid = (N // R,)
  nchunks = 0 if b is None else len(chunks)
  if b is None:
    b_arr = jnp.zeros((1, din), jnp.float32)
  else:
    b_arr = b.reshape(1, din)
  in_specs = (
      [pl.BlockSpec((R, c.shape[1]), lambda i: (i, 0)) for c in chunks]
      + [
          pl.BlockSpec((1, din), lambda i: (0, 0)),
          pl.BlockSpec((din, dout), lambda i: (0, 0)),
          pl.BlockSpec((dout, 1), lambda i: (0, 0)),
          pl.BlockSpec((dout, 1), lambda i: (0, 0)),
      ]
  )
  out_specs = [
      pl.BlockSpec((R, dout), lambda i: (i, 0)),
      pl.BlockSpec((R, 1), lambda i: (i, 0)),
      pl.BlockSpec((R, 1), lambda i: (i, 0)),
      pl.BlockSpec((1, 1), lambda i: (0, 0)),
  ]
  out_shape = [
      jax.ShapeDtypeStruct((N, dout), jnp.float32),
      jax.ShapeDtypeStruct((N, 1), jnp.float32),
      jax.ShapeDtypeStruct((N, 1), jnp.float32),
      jax.ShapeDtypeStruct((1, 1), jnp.float32),
  ]
  body = functools.partial(_tc_body, nchunks)
  return pl.pallas_call(
      body,
      grid=grid,
      in_specs=in_specs,
      out_specs=out_specs,
      out_shape=out_shape,
  )(*chunks, b_arr, w, asv.reshape(dout, 1), adv.reshape(dout, 1))


def _tc_final(parts, b3, batch_index):
  def body(parts_ref, b_ref, bi_ref, pooled_ref, acc_ref):
    i = pl.program_id(0)

    @pl.when(i == 0)
    def _():
      acc_ref[...] = parts_ref[0]

    @pl.when(i > 0)
    def _():
      acc_ref[...] = acc_ref[...] + parts_ref[0]

    @pl.when(i == NW - 1)
    def _():
      h3 = jax.nn.relu(acc_ref[:, :N] + b_ref[...])
      gids = lax.broadcasted_iota(jnp.int32, (NUM_GRAPHS, N), 0)
      onehot = jnp.where(bi_ref[...] == gids, 1.0, 0.0)
      sums = lax.dot_general(
          onehot, h3, (((1,), (1,)), ((), ())),
          preferred_element_type=jnp.float32)
      counts = jnp.sum(onehot, axis=1, keepdims=True)
      pooled_ref[...] = sums / jnp.maximum(counts, 1.0)

  return pl.pallas_call(
      body,
      grid=(NW,),
      in_specs=[
          pl.BlockSpec((1, 4, NP), lambda i: (i, 0, 0)),
          pl.BlockSpec((4, 1), lambda i: (0, 0)),
          pl.BlockSpec((1, N), lambda i: (0, 0)),
      ],
      out_specs=pl.BlockSpec((NUM_GRAPHS, 4), lambda i: (0, 0)),
      out_shape=jax.ShapeDtypeStruct((NUM_GRAPHS, 4), jnp.float32),
      scratch_shapes=[pltpu.VMEM((4, NP), jnp.float32)],
  )(parts, b3.reshape(4, 1), batch_index.reshape(1, N))


def _sc_edge_p(src2d, dst2d, als, ald, amax16, irows):
  @functools.partial(
      pl.kernel,
      out_type=(
          jax.ShapeDtypeStruct((ROWS, 128), jnp.float32),
          jax.ShapeDtypeStruct((NC * SR, 128), jnp.float32),
      ),
      mesh=_mesh(),
      compiler_params=pltpu.CompilerParams(needs_layout_passes=False),
      scratch_types=dict(
          als_t=pltpu.VMEM((NP,), jnp.float32),
          ald_t=pltpu.VMEM((NP,), jnp.float32),
          am_t=pltpu.VMEM((16,), jnp.float32),
          s_t=pltpu.VMEM((SR, 128), jnp.float32),
          src_t=pltpu.VMEM((RPT_A, 128), jnp.int32),
          dst_t=pltpu.VMEM((RPT_A, 128), jnp.int32),
          p_t=pltpu.VMEM((RPT_A, 128), jnp.float32),
          irows_t=pltpu.VMEM((1, SR), jnp.int32),
          z8_t=pltpu.VMEM((8, 128), jnp.float32),
          s_sh=pltpu.VMEM_SHARED((SR, 128), jnp.float32),
      ),
  )
  def k(src_h, dst_h, als_h, ald_h, am_h, irows_h, p_h, sparts_h, *, als_t,
        ald_t, am_t, s_t, src_t, dst_t, p_t, irows_t, z8_t, s_sh):
    cid = lax.axis_index("c")
    sid = lax.axis_index("s")
    wid = sid * NC + cid

    pltpu.sync_copy(als_h, als_t)
    pltpu.sync_copy(ald_h, ald_t)
    pltpu.sync_copy(am_h, am_t)
    pltpu.sync_copy(irows_h, irows_t)
    pltpu.sync_copy(src_h.at[pl.ds(wid * RPT_A, RPT_A)], src_t)
    pltpu.sync_copy(dst_h.at[pl.ds(wid * RPT_A, RPT_A)], dst_t)

    zv = jnp.zeros((L,), jnp.float32)

    def zloop(i, _):
      s_t[i >> 3, pl.ds((i & 7) * L, L)] = zv
      return 0

    lax.fori_loop(0, SR * 8, zloop, 0)

    for r8 in range(8):
      for j in range(128 // L):
        z8_t[r8, pl.ds(j * L, L)] = zv

    @pl.when(sid < SR // 8)
    def _():
      pltpu.sync_copy(z8_t, s_sh.at[pl.ds(sid * 8, 8)])

    am = am_t[pl.ds(0, L)][0]

    def row(g, _):
      @pl.when(wid * RPT_A + g < LIVE_ROWS)
      def _():
        def grp(j, _):
          sl = pl.ds(j * L, L)
          sv = src_t[g, sl]
          dv = dst_t[g, sl]
          a1 = plsc.load_gather(als_t, [sv])
          a2 = plsc.load_gather(ald_t, [dv])
          e = _leaky(a1 + a2)
          mb = _leaky(am + a2)
          pv = jnp.exp(e - mb)
          p_t[g, sl] = pv
          plsc.addupdate_scatter(s_t, [dv >> 7, dv & 127], pv)
          return 0

        lax.fori_loop(0, 128 // L, grp, 0)
      return 0

    lax.fori_loop(0, RPT_A, row, 0)
    pltpu.sync_copy(p_t, p_h.at[pl.ds(wid * RPT_A, RPT_A)])

    plsc.subcore_barrier()
    pltpu.sync_copy(s_t, s_sh.at[irows_t.at[0]], add=True)
    plsc.subcore_barrier()

    @pl.when(sid < SR // 8)
    def _():
      pltpu.sync_copy(
          s_sh.at[pl.ds(sid * 8, 8)],
          sparts_h.at[pl.ds(cid * SR + sid * 8, 8)])

  return k(src2d, dst2d, als, ald, amax16, irows)


def _sc_aggregate(src64, dst64, p64, s_parts, h_chunks):
  nch = len(h_chunks)

  @functools.partial(
      pl.kernel,
      out_type=tuple(
          jax.ShapeDtypeStruct((NP, 128), jnp.float32) for _ in range(nch)),
      mesh=_mesh(),
      compiler_params=pltpu.CompilerParams(needs_layout_passes=False),
      scratch_types=dict(
          s_t=pltpu.VMEM((SR, 128), jnp.float32),
          src_t=pltpu.VMEM((HSB, 64), jnp.int32),
          dst_t=pltpu.VMEM((HSB, 64), jnp.int32),
          a_t=pltpu.VMEM((HSB, 64), jnp.float32),
          rows0=pltpu.VMEM((64, 128), jnp.float32),
          rows1=pltpu.VMEM((64, 128), jnp.float32),
          rows2=pltpu.VMEM((64, 128), jnp.float32),
          rows3=pltpu.VMEM((64, 128), jnp.float32),
          gsem0=pltpu.SemaphoreType.DMA,
          gsem1=pltpu.SemaphoreType.DMA,
          gsem2=pltpu.SemaphoreType.DMA,
          gsem3=pltpu.SemaphoreType.DMA,
          ssem0=pltpu.SemaphoreType.DMA,
          ssem1=pltpu.SemaphoreType.DMA,
          ssem2=pltpu.SemaphoreType.DMA,
          ssem3=pltpu.SemaphoreType.DMA,
          acc_sh=pltpu.VMEM_SHARED((NP, 128), jnp.float32),
      ),
  )
  def k(src_h, dst_h, p_h, sparts_h, *rest, s_t, src_t, dst_t, a_t, rows0,
        rows1, rows2, rows3, gsem0, gsem1, gsem2, gsem3, ssem0, ssem1,
        ssem2, ssem3, acc_sh):
    h_hs = rest[:nch]
    out_hs = rest[nch:]
    cid = lax.axis_index("c")
    sid = lax.axis_index("s")
    rows = [rows0, rows1, rows2, rows3]
    gsems = [gsem0, gsem1, gsem2, gsem3]
    ssems = [ssem0, ssem1, ssem2, ssem3]

    pltpu.sync_copy(sparts_h.at[pl.ds(0, SR)], s_t)
    pltpu.sync_copy(sparts_h.at[pl.ds(SR, 64)], rows0)
    pltpu.sync_copy(sparts_h.at[pl.ds(SR + 64, SR - 64)],
                    rows1.at[pl.ds(0, SR - 64)])

    def addl(i, _):
      g = i >> 3
      sl = pl.ds((i & 7) * L, L)
      s_t[g, sl] = s_t[g, sl] + rows0[g, sl]
      return 0

    lax.fori_loop(0, 64 * 8, addl, 0)

    def addl2(i, _):
      g = i >> 3
      sl = pl.ds((i & 7) * L, L)
      s_t[64 + g, sl] = s_t[64 + g, sl] + rows1[g, sl]
      return 0

    lax.fori_loop(0, (SR - 64) * 8, addl2, 0)

    zv = jnp.zeros((L,), jnp.float32)

    def _scale(buf, g):
      def scale(j16, _):
        avv = a_t[g, pl.ds(j16 * L, L)]
        for kk in range(L):
          av = jnp.full((L,), avv[kk], jnp.float32)
          r = j16 * L + kk
          for j in range(128 // L):
            sl = pl.ds(j * L, L)
            buf[r, sl] = buf[r, sl] * av
        return 0

      lax.fori_loop(0, 64 // L, scale, 0)

    for c in range(nch):

      @pl.when(c % NC == cid)
      def _():
        def zrow(i, _):
          for j in range(128 // L):
            rows0[i, pl.ds(j * L, L)] = zv
          return 0

        lax.fori_loop(0, 64, zrow, 0)

        def zacc(i, _):
          pltpu.sync_copy(rows0,
                          acc_sh.at[pl.ds((sid * (NPT // 64) + i) * 64, 64)])
          return 0

        lax.fori_loop(0, NPT // 64, zacc, 0)
        plsc.subcore_barrier()

        def block(bb, _):
          row0 = sid * HPT + bb * HSB

          @pl.when(row0 < LIVE_H)
          def _():
            pltpu.sync_copy(src_h.at[pl.ds(row0, HSB)], src_t)
            pltpu.sync_copy(dst_h.at[pl.ds(row0, HSB)], dst_t)
            pltpu.sync_copy(p_h.at[pl.ds(row0, HSB)], a_t)

            def arow(g, _):
              for j in range(64 // L):
                sl = pl.ds(j * L, L)
                dv = dst_t[g, sl]
                sg = plsc.load_gather(s_t, [dv >> 7, dv & 127])
                a_t[g, sl] = a_t[g, sl] / (sg + 1e-16)
              return 0

            lax.fori_loop(0, HSB, arow, 0)

          @pl.when(row0 + HSB <= LIVE_H)
          def _():
            for u in range(3):
              pltpu.async_copy(h_hs[c].at[src_t.at[u]], rows[u], gsems[u])

            def quad(i8, _):
              for u in range(4):
                r = i8 * 4 + u
                pltpu.make_async_copy(
                    h_hs[c].at[src_t.at[r]], rows[u], gsems[u]).wait()
                _scale(rows[u], r)
                pltpu.async_copy(rows[u], acc_sh.at[dst_t.at[r]],
                                 ssems[u], add=True)
                w = (u + 3) % 4

                def _refill(r=r, u=u, w=w):
                  pltpu.make_async_copy(rows[w], acc_sh.at[dst_t.at[r]],
                                        ssems[w]).wait()
                  pltpu.async_copy(h_hs[c].at[src_t.at[r + 3]], rows[w],
                                   gsems[w])

                if u == 0:
                  @pl.when(i8 > 0)
                  def _():
                    pltpu.make_async_copy(rows[3], acc_sh.at[dst_t.at[r]],
                                          ssems[3]).wait()

                  pltpu.async_copy(h_hs[c].at[src_t.at[r + 3]], rows[3],
                                   gsems[3])
                else:
                  @pl.when(i8 < (HSB // 4) - 1)
                  def _():
                    _refill()
              return 0

            lax.fori_loop(0, HSB // 4, quad, 0)
            for u in range(4):
              pltpu.make_async_copy(rows[u], acc_sh.at[dst_t.at[u]],
                                    ssems[u]).wait()

          @pl.when(jnp.logical_and(row0 < LIVE_H, row0 + HSB > LIVE_H))
          def _():
            def srow(g, _):
              @pl.when(row0 + g < LIVE_H)
              def _():
                pltpu.async_copy(h_hs[c].at[src_t.at[g]], rows0,
                                 gsem0).wait()
                _scale(rows0, g)
                pltpu.sync_copy(rows0, acc_sh.at[dst_t.at[g]], add=True)
              return 0

            lax.fori_loop(0, HSB, srow, 0)
          return 0

        lax.fori_loop(0, HPT // HSB, block, 0)
        plsc.subcore_barrier()
        pltpu.sync_copy(
            acc_sh.at[pl.ds(sid * NPT, NPT)],
            out_hs[c].at[pl.ds(sid * NPT, NPT)])
        plsc.subcore_barrier()

  return k(src64, dst64, p64, s_parts, *h_chunks)


def _sc_aggregate_small(src2d, dst2d, p2d, s_parts, h3r):
  @functools.partial(
      pl.kernel,
      out_type=jax.ShapeDtypeStruct((NW, 4 * SR, 128), jnp.float32),
      mesh=_mesh(),
      compiler_params=pltpu.CompilerParams(needs_layout_passes=False),
      scratch_types=dict(
          s_t=pltpu.VMEM((SR, 128), jnp.float32),
          h_t=pltpu.VMEM((4 * SR, 128), jnp.float32),
          acc_t=pltpu.VMEM((4 * SR, 128), jnp.float32),
          src_t=pltpu.VMEM((RPT_A, 128), jnp.int32),
          dst_t=pltpu.VMEM((RPT_A, 128), jnp.int32),
          p_t=pltpu.VMEM((RPT_A, 128), jnp.float32),
      ),
  )
  def k(src_h, dst_h, p_h, sparts_h, h3_h, parts_h, *, s_t, h_t, acc_t,
        src_t, dst_t, p_t):
    cid = lax.axis_index("c")
    sid = lax.axis_index("s")
    wid = sid * NC + cid
    pltpu.sync_copy(sparts_h.at[pl.ds(0, SR)], s_t)
    pltpu.sync_copy(sparts_h.at[pl.ds(SR, SR)], h_t.at[pl.ds(0, SR)])

    def addl(i, _):
      g = i >> 3
      sl = pl.ds((i & 7) * L, L)
      s_t[g, sl] = s_t[g, sl] + h_t[g, sl]
      return 0

    lax.fori_loop(0, SR * 8, addl, 0)

    pltpu.sync_copy(h3_h, h_t)
    pltpu.sync_copy(src_h.at[pl.ds(wid * RPT_A, RPT_A)], src_t)
    pltpu.sync_copy(dst_h.at[pl.ds(wid * RPT_A, RPT_A)], dst_t)
    pltpu.sync_copy(p_h.at[pl.ds(wid * RPT_A, RPT_A)], p_t)

    zv = jnp.zeros((L,), jnp.float32)

    def zloop(i, _):
      acc_t[i >> 3, pl.ds((i & 7) * L, L)] = zv
      return 0

    lax.fori_loop(0, 4 * SR * 8, zloop, 0)

    def row(g, _):
      @pl.when(wid * RPT_A + g < LIVE_ROWS)
      def _():
        def grp(j, _):
          sl = pl.ds(j * L, L)
          sv = src_t[g, sl]
          dv = dst_t[g, sl]
          dv_hi = dv >> 7
          dv_lo = dv & 127
          sv_hi = sv >> 7
          sv_lo = sv & 127
          sg = plsc.load_gather(s_t, [dv_hi, dv_lo])
          av = p_t[g, sl] / (sg + 1e-16)
          for col in range(4):
            hv = plsc.load_gather(h_t, [sv_hi + (col * SR), sv_lo])
            plsc.addupdate_scatter(acc_t, [dv_hi + (col * SR), dv_lo],
                                   hv * av)
          return 0

        lax.fori_loop(0, 128 // L, grp, 0)
      return 0

    lax.fori_loop(0, RPT_A, row, 0)
    pltpu.sync_copy(acc_t, parts_h.at[wid])

  return k(src2d, dst2d, p2d, s_parts, h3r)


def kernel(x, edge_index, batch_index, W1, as1, ad1, b1, W2, as2, ad2, b2,
           W3, as3, ad3, b3):
  loops = jnp.arange(N, dtype=edge_index.dtype)
  src = jnp.concatenate([edge_index[0], loops])
  dst = jnp.concatenate([edge_index[1], loops])
  npad = EP - (E + N)
  src = jnp.concatenate([src, jnp.zeros((npad,), jnp.int32)])
  dst = jnp.concatenate([dst, jnp.full((npad,), N, jnp.int32)])
  src2d = src.reshape(ROWS, 128)
  dst2d = dst.reshape(ROWS, 128)
  src64 = src.reshape(HROWS, 64)
  dst64 = dst.reshape(HROWS, 64)
  irows = jnp.arange(SR, dtype=jnp.int32).reshape(1, SR)

  def pad_nodes(v):
    return jnp.concatenate([v.reshape(N), jnp.zeros((NP - N,), jnp.float32)])

  def edge_phase(als, ald, amax):
    als_p = pad_nodes(als)
    ald_p = pad_nodes(ald)
    am16 = jnp.broadcast_to(amax.reshape(1), (16,))
    return _sc_edge_p(src2d, dst2d, als_p, ald_p, am16, irows)

  h1, als1, ald1, am1 = _tc_layer([x], None, W1, as1, ad1)
  p1, s1 = edge_phase(als1, ald1, am1)
  h1_chunks = [h1[:, c * 128:(c + 1) * 128] for c in range(2)]
  o1 = _sc_aggregate(src64, dst64, p1.reshape(HROWS, 64), s1, h1_chunks)
  o1 = [o[:N] for o in o1]

  h2, als2, ald2, am2 = _tc_layer(o1, b1, W2, as2, ad2)
  p2, s2 = edge_phase(als2, ald2, am2)
  h2_chunks = [h2[:, c * 128:(c + 1) * 128] for c in range(4)]
  o2 = _sc_aggregate(src64, dst64, p2.reshape(HROWS, 64), s2, h2_chunks)
  o2 = [o[:N] for o in o2]

  h3, als3, ald3, am3 = _tc_layer(o2, b2, W3, as3, ad3)
  p3, s3 = edge_phase(als3, ald3, am3)
  h3r = jnp.pad(h3, ((0, NP - N), (0, 0))).T.reshape(4 * SR, 128)
  parts = _sc_aggregate_small(src2d, dst2d, p3, s3, h3r)
  parts = parts.reshape(NW, 4, NP)

  return _tc_final(parts, b3, batch_index)
